# baseline (speedup 1.0000x reference)
import functools

import jax
import jax.numpy as jnp
from jax import lax
from jax.experimental import pallas as pl
from jax.experimental.pallas import tpu as pltpu
from jax.experimental.pallas import tpu_sc as plsc

EPS = 1e-09


def _make_sc_gather(num_tokens, dim, chunk):
    info = plsc.get_sparse_core_info()
    nc, ns = info.num_cores, info.num_subcores
    nw = nc * ns
    per_w = num_tokens // nw
    n_chunks = per_w // chunk
    mesh = plsc.VectorSubcoreMesh(core_axis_name="c", subcore_axis_name="s")

    @functools.partial(
        pl.kernel,
        out_type=jax.ShapeDtypeStruct((num_tokens, dim), jnp.float32),
        mesh=mesh,
        scratch_types=[
            pltpu.VMEM((per_w,), jnp.int32),
            pltpu.VMEM((chunk, dim), jnp.float32),
            pltpu.VMEM((chunk, dim), jnp.float32),
            pltpu.SemaphoreType.DMA,
            pltpu.SemaphoreType.DMA,
        ],
    )
    def sc_gather(ids_hbm, table_hbm, out_hbm, idx_v, buf0, buf1, sem0, sem1):
        wid = lax.axis_index("s") * nc + lax.axis_index("c")
        base = wid * per_w
        pltpu.sync_copy(ids_hbm.at[pl.ds(base, per_w)], idx_v)
        bufs = (buf0, buf1)
        sems = (sem0, sem1)
        copies = [None] * n_chunks
        for c in range(n_chunks):
            copies[c] = pltpu.async_copy(
                table_hbm.at[idx_v.at[pl.ds(c * chunk, chunk)]],
                bufs[c % 2],
                sems[c % 2],
            )
            if c >= 1:
                copies[c - 1].wait()
                pltpu.sync_copy(
                    bufs[(c - 1) % 2],
                    out_hbm.at[pl.ds(base + (c - 1) * chunk, chunk)],
                )
        copies[n_chunks - 1].wait()
        pltpu.sync_copy(
            bufs[(n_chunks - 1) % 2],
            out_hbm.at[pl.ds(base + (n_chunks - 1) * chunk, chunk)],
        )

    return sc_gather


def _tc_ln_body(w_ref, p_ref, g_ref, b_ref, out_ref):
    w = w_ref[0]
    p = p_ref[...]
    x = w + p
    mean = jnp.mean(x, axis=-1, keepdims=True)
    xc = x - mean
    var = jnp.mean(xc * xc, axis=-1, keepdims=True)
    xhat = xc * lax.rsqrt(var + EPS)
    out_ref[0] = xhat * g_ref[...] + b_ref[...]


def _tc_ln(w3, pos_table, gamma, beta, sblk):
    b, n, d = w3.shape
    grid = (n // sblk, b)
    return pl.pallas_call(
        _tc_ln_body,
        grid=grid,
        in_specs=[
            pl.BlockSpec((1, sblk, d), lambda j, i: (i, j, 0)),
            pl.BlockSpec((sblk, d), lambda j, i: (j, 0)),
            pl.BlockSpec((1, d), lambda j, i: (0, 0)),
            pl.BlockSpec((1, d), lambda j, i: (0, 0)),
        ],
        out_specs=pl.BlockSpec((1, sblk, d), lambda j, i: (i, j, 0)),
        out_shape=jax.ShapeDtypeStruct((b, n, d), jnp.float32),
    )(w3, pos_table, gamma.reshape(1, d), beta.reshape(1, d))


def _tc_pos_body(p_ref, out_ref):
    p = p_ref[...]
    out_ref[...] = jnp.broadcast_to(p[None], out_ref.shape)


def _tc_pos(pos_table, b, sblk):
    n, d = pos_table.shape
    return pl.pallas_call(
        _tc_pos_body,
        grid=(n // sblk,),
        in_specs=[pl.BlockSpec((sblk, d), lambda j: (j, 0))],
        out_specs=pl.BlockSpec((b, sblk, d), lambda j: (0, j, 0)),
        out_shape=jax.ShapeDtypeStruct((b, n, d), jnp.float32),
    )(pos_table)


def kernel(input_ids, word_table, pos_table, ln_gamma, ln_beta):
    b, n = input_ids.shape
    d = word_table.shape[1]
    ids = input_ids.reshape(-1).astype(jnp.int32)
    gathered = _make_sc_gather(b * n, d, 64)(ids, word_table)
    pos_out = _tc_pos(pos_table, b, 512)
    w3 = gathered.reshape(b, n, d)
    out = _tc_ln(w3, pos_table, ln_gamma, ln_beta, 2048)
    return out, pos_out


# speedup vs baseline: 1.0161x; 1.0161x over previous
import functools

import jax
import jax.numpy as jnp
from jax import lax
from jax.experimental import pallas as pl
from jax.experimental.pallas import tpu as pltpu
from jax.experimental.pallas import tpu_sc as plsc

EPS = 1e-09


def _make_sc_gather(num_tokens, dim, chunk):
    info = plsc.get_sparse_core_info()
    nc, ns = info.num_cores, info.num_subcores
    nw = nc * ns
    per_w = num_tokens // nw
    n_chunks = per_w // chunk
    mesh = plsc.VectorSubcoreMesh(core_axis_name="c", subcore_axis_name="s")

    @functools.partial(
        pl.kernel,
        out_type=jax.ShapeDtypeStruct((num_tokens, dim), jnp.float32),
        mesh=mesh,
        scratch_types=[
            pltpu.VMEM((per_w,), jnp.int32),
            pltpu.VMEM((chunk, dim), jnp.float32),
            pltpu.VMEM((chunk, dim), jnp.float32),
            pltpu.SemaphoreType.DMA,
            pltpu.SemaphoreType.DMA,
        ],
    )
    def sc_gather(ids_hbm, table_hbm, out_hbm, idx_v, buf0, buf1, sem0, sem1):
        wid = lax.axis_index("s") * nc + lax.axis_index("c")
        base = wid * per_w
        pltpu.sync_copy(ids_hbm.at[pl.ds(base, per_w)], idx_v)
        bufs = (buf0, buf1)
        sems = (sem0, sem1)
        copies = [None] * n_chunks
        for c in range(n_chunks):
            copies[c] = pltpu.async_copy(
                table_hbm.at[idx_v.at[pl.ds(c * chunk, chunk)]],
                bufs[c % 2],
                sems[c % 2],
            )
            if c >= 1:
                copies[c - 1].wait()
                pltpu.sync_copy(
                    bufs[(c - 1) % 2],
                    out_hbm.at[pl.ds(base + (c - 1) * chunk, chunk)],
                )
        copies[n_chunks - 1].wait()
        pltpu.sync_copy(
            bufs[(n_chunks - 1) % 2],
            out_hbm.at[pl.ds(base + (n_chunks - 1) * chunk, chunk)],
        )

    return sc_gather


def _tc_ln_body(w_ref, p_ref, g_ref, b_ref, out_ref):
    w = w_ref[0]
    p = p_ref[...]
    x = w + p
    mean = jnp.mean(x, axis=-1, keepdims=True)
    xc = x - mean
    var = jnp.mean(xc * xc, axis=-1, keepdims=True)
    xhat = xc * lax.rsqrt(var + EPS)
    out_ref[0] = xhat * g_ref[...] + b_ref[...]


def _tc_ln(w3, pos_table, gamma, beta, sblk):
    b, n, d = w3.shape
    grid = (n // sblk, b)
    return pl.pallas_call(
        _tc_ln_body,
        grid=grid,
        in_specs=[
            pl.BlockSpec((1, sblk, d), lambda j, i: (i, j, 0)),
            pl.BlockSpec((sblk, d), lambda j, i: (j, 0)),
            pl.BlockSpec((1, d), lambda j, i: (0, 0)),
            pl.BlockSpec((1, d), lambda j, i: (0, 0)),
        ],
        out_specs=pl.BlockSpec((1, sblk, d), lambda j, i: (i, j, 0)),
        out_shape=jax.ShapeDtypeStruct((b, n, d), jnp.float32),
    )(w3, pos_table, gamma.reshape(1, d), beta.reshape(1, d))


def _tc_pos_body(p_ref, out_ref):
    p = p_ref[...]
    out_ref[...] = jnp.broadcast_to(p[None], out_ref.shape)


def _tc_pos(pos_table, b, sblk):
    n, d = pos_table.shape
    return pl.pallas_call(
        _tc_pos_body,
        grid=(n // sblk,),
        in_specs=[pl.BlockSpec((sblk, d), lambda j: (j, 0))],
        out_specs=pl.BlockSpec((b, sblk, d), lambda j: (0, j, 0)),
        out_shape=jax.ShapeDtypeStruct((b, n, d), jnp.float32),
    )(pos_table)


def kernel(input_ids, word_table, pos_table, ln_gamma, ln_beta):
    b, n = input_ids.shape
    d = word_table.shape[1]
    ids = input_ids.reshape(-1).astype(jnp.int32)
    gathered = _make_sc_gather(b * n, d, 64)(ids, word_table)
    pos_out = _tc_pos(pos_table, b, 2048)
    w3 = gathered.reshape(b, n, d)
    out = _tc_ln(w3, pos_table, ln_gamma, ln_beta, 2048)
    return out, pos_out
